# baseline (speedup 1.0000x reference)
import functools

import jax
import jax.numpy as jnp
from jax import lax
from jax.experimental import pallas as pl
from jax.experimental.pallas import tpu as pltpu
from jax.experimental.pallas import tpu_sc as plsc

_N = 10000
_E = 320000
_DIN = 128
_DH = 64
_DOUT = 128

_NC = 2
_NS = 16
_NT = _NC * _NS
_CH = 128
_NCHK = 80
_PER_TILE = _CH * _NCHK
_EPAD = _NT * _PER_TILE
_NACC = 10240
_RPT = _NACC // _NS
_SRT = 128
_DEGW = 16
_NBUF = 4
_GD = 3
_SL = _NBUF - _GD


@functools.lru_cache(maxsize=None)
def _make_seg_sum(with_deg):
    mesh = plsc.VectorSubcoreMesh(core_axis_name="c", subcore_axis_name="s")
    out_types = [jax.ShapeDtypeStruct((_NC * _NACC, _DH), jnp.float32)]
    scratch = (
        [pltpu.VMEM((_NCHK, _CH), jnp.int32)] * 2 +
        [pltpu.VMEM((_CH, _DH), jnp.float32)] * _NBUF +
        [pltpu.VMEM((_SRT, _DH), jnp.float32)] +
        [pltpu.VMEM_SHARED((_NACC, _DH), jnp.float32)] +
        [pltpu.SemaphoreType.DMA] * (2 * _NBUF)
    )
    if with_deg:
        out_types.append(jax.ShapeDtypeStruct((_NC * _NACC, _DEGW),
                                              jnp.float32))
        scratch += [
            pltpu.VMEM((_CH, _DEGW), jnp.float32),
            pltpu.VMEM((_SRT, _DEGW), jnp.float32),
            pltpu.VMEM_SHARED((_NACC, _DEGW), jnp.float32),
        ]

    @functools.partial(
        pl.kernel, mesh=mesh, out_type=out_types, scratch_types=scratch,
        compiler_params=pltpu.CompilerParams(use_tc_tiling_on_sc=False))
    def seg(vals, edge_r, *rest):
        if with_deg:
            (out, outd, src_v, dst_v, r0, r1, r2, r3, zbuf, acc,
             g0, g1, g2, g3, s0, s1, s2, s3,
             ones_v, zbufd, accd) = rest
        else:
            (out, src_v, dst_v, r0, r1, r2, r3, zbuf, acc,
             g0, g1, g2, g3, s0, s1, s2, s3) = rest
        rbufs = (r0, r1, r2, r3)
        gsems = (g0, g1, g2, g3)
        ssems = (s0, s1, s2, s3)
        c = lax.axis_index("c")
        s = lax.axis_index("s")
        tid = c * _NS + s
        rowb = s * _RPT

        zero16 = jnp.zeros((16,), jnp.float32)

        pltpu.async_copy(edge_r.at[tid], src_v, gsems[0])
        pltpu.async_copy(edge_r.at[_NT + tid], dst_v, gsems[1])

        def zrow(i, _):
            for j in range(_DH // 16):
                zbuf[i, pl.ds(j * 16, 16)] = zero16
            return 0
        lax.fori_loop(0, _SRT, zrow, 0)
        for k in range(_RPT // _SRT):
            pltpu.async_copy(zbuf, acc.at[pl.ds(rowb + k * _SRT, _SRT)],
                             ssems[0])

        if with_deg:
            def zdrow(i, _):
                zbufd[i, pl.ds(0, 16)] = zero16
                return 0
            lax.fori_loop(0, _SRT, zdrow, 0)
            for k in range(_RPT // _SRT):
                pltpu.async_copy(zbufd, accd.at[pl.ds(rowb + k * _SRT, _SRT)],
                                 ssems[1])

            one16 = jnp.full((16,), 1.0, jnp.float32)

            def orow(i, _):
                ones_v[i, pl.ds(0, 16)] = one16
                return 0
            lax.fori_loop(0, _CH, orow, 0)

        for k in range(_RPT // _SRT):
            pltpu.make_async_copy(zbuf, acc.at[pl.ds(rowb, _SRT)],
                                  ssems[0]).wait()
            if with_deg:
                pltpu.make_async_copy(zbufd, accd.at[pl.ds(rowb, _SRT)],
                                      ssems[1]).wait()
        pltpu.make_async_copy(edge_r.at[tid], src_v, gsems[0]).wait()
        pltpu.make_async_copy(edge_r.at[_NT + tid], dst_v, gsems[1]).wait()

        plsc.subcore_barrier()

        def body(j, b, wait_sc, fire_g):
            nb = (b + _GD) % _NBUF
            pltpu.make_async_copy(vals.at[src_v.at[j]], rbufs[b],
                                  gsems[b]).wait()
            pltpu.async_copy(rbufs[b], acc.at[dst_v.at[j]], ssems[b],
                             add=True)
            if with_deg:
                pltpu.async_copy(ones_v, accd.at[dst_v.at[j]], ssems[b],
                                 add=True)
            if wait_sc:
                pltpu.make_async_copy(rbufs[nb], acc.at[dst_v.at[j]],
                                      ssems[nb]).wait()
                if with_deg:
                    pltpu.make_async_copy(ones_v, accd.at[dst_v.at[j]],
                                          ssems[nb]).wait()
            if fire_g:
                pltpu.async_copy(vals.at[src_v.at[j + _GD]], rbufs[nb],
                                 gsems[nb])

        for g in range(_GD):
            pltpu.async_copy(vals.at[src_v.at[g]], rbufs[g], gsems[g])

        for j in range(_NBUF):
            body(j, j, j >= _SL, j + _GD < _NCHK)

        def mid_chunk(jj, _):
            j0 = jj * _NBUF
            for b in range(_NBUF):
                body(j0 + b, b, True, True)
            return 0
        lax.fori_loop(1, _NCHK // _NBUF - 1, mid_chunk, 0)

        for j in range(_NCHK - _NBUF, _NCHK):
            body(j, j % _NBUF, True, j + _GD < _NCHK)

        for jj in range(_NCHK - _SL, _NCHK):
            pltpu.make_async_copy(rbufs[jj % _NBUF], acc.at[dst_v.at[0]],
                                  ssems[jj % _NBUF]).wait()
            if with_deg:
                pltpu.make_async_copy(ones_v, accd.at[dst_v.at[0]],
                                      ssems[jj % _NBUF]).wait()

        plsc.subcore_barrier()

        ob = c * _NACC + rowb
        pltpu.async_copy(acc.at[pl.ds(rowb, _RPT)],
                         out.at[pl.ds(ob, _RPT)], gsems[0])
        if with_deg:
            pltpu.async_copy(accd.at[pl.ds(rowb, _RPT)],
                             outd.at[pl.ds(ob, _RPT)], gsems[1])
            pltpu.make_async_copy(accd.at[pl.ds(rowb, _RPT)],
                                  outd.at[pl.ds(ob, _RPT)], gsems[1]).wait()
        pltpu.make_async_copy(acc.at[pl.ds(rowb, _RPT)],
                              out.at[pl.ds(ob, _RPT)], gsems[0]).wait()

    return seg


def _tc_proj(x, W):
    def body(x_ref, w_ref, y_ref):
        y_ref[...] = jnp.dot(x_ref[...], w_ref[...],
                             preferred_element_type=jnp.float32)

    return pl.pallas_call(
        body,
        out_shape=jax.ShapeDtypeStruct((x.shape[0], W.shape[1]), jnp.float32),
    )(x, W)


def _tc_mid(p1, pd, z1, b1):
    def body(p_ref, pd_ref, z1_ref, b1_ref, h_ref, d_ref):
        agg = p_ref[0:_N, :] + p_ref[_NACC:_NACC + _N, :]
        deg = pd_ref[0:_N, 0:1] + pd_ref[_NACC:_NACC + _N, 0:1]
        degc = jnp.maximum(deg, 1.0)
        mean = agg / degc
        h = jnp.maximum(mean + b1_ref[...] + z1_ref[...], 0.0)
        h_ref[...] = h
        d_ref[...] = jnp.broadcast_to(degc, (_N, 8))

    return pl.pallas_call(
        body,
        out_shape=[jax.ShapeDtypeStruct((_N, _DH), jnp.float32),
                   jax.ShapeDtypeStruct((_N, 8), jnp.float32)],
    )(p1, pd, z1, b1)


def _tc_post(p2, degc, z2, b2, W2l):
    def body(p_ref, d_ref, z2_ref, b2_ref, w_ref, o_ref):
        agg = p_ref[0:_N, :] + p_ref[_NACC:_NACC + _N, :]
        mean = agg / d_ref[:, 0:1]
        o_ref[...] = (jnp.dot(mean, w_ref[...], preferred_element_type=jnp.float32)
                      + b2_ref[...] + z2_ref[...])

    return pl.pallas_call(
        body,
        out_shape=jax.ShapeDtypeStruct((_N, _DOUT), jnp.float32),
    )(p2, degc, z2, b2, W2l)


def _first(res):
    return res[0] if isinstance(res, (list, tuple)) else res


def kernel(x, edge_index, W1l, b1, W1r, W2l, b2, W2r):
    ar = jnp.arange(_EPAD - _E, dtype=jnp.int32) % 16
    padblk = jnp.stack([ar, _N + ar])
    edge_r = jnp.concatenate([edge_index, padblk], axis=1).reshape(
        2 * _NT, _NCHK, _CH)

    y1 = _tc_proj(x, W1l)
    p1, pd = _make_seg_sum(True)(y1, edge_r)
    z1 = _tc_proj(x, W1r)
    h, degc = _tc_mid(p1, pd, z1, b1.reshape(1, _DH))
    p2 = _first(_make_seg_sum(False)(h, edge_r))
    z2 = _tc_proj(h, W2r)
    out = _tc_post(p2, degc, z2, b2.reshape(1, _DOUT), W2l)
    return out


# speedup vs baseline: 1.0891x; 1.0891x over previous
import functools

import jax
import jax.numpy as jnp
from jax import lax
from jax.experimental import pallas as pl
from jax.experimental.pallas import tpu as pltpu
from jax.experimental.pallas import tpu_sc as plsc

_N = 10000
_E = 320000
_DIN = 128
_DH = 64
_DOUT = 128

_NC = 2
_NS = 16
_NT = _NC * _NS
_CH = 128
_NCHK = 79
_PER_TILE = _CH * _NCHK
_EPAD = _NT * _PER_TILE
_NACC = 10240
_RPT = _NACC // _NS
_SRT = 128
_DEGW = 16
_NBUF = 5
_GD = 3
_SL = _NBUF - _GD


@functools.lru_cache(maxsize=None)
def _make_seg_sum(with_deg):
    mesh = plsc.VectorSubcoreMesh(core_axis_name="c", subcore_axis_name="s")
    out_types = [jax.ShapeDtypeStruct((_NC * _NACC, _DH), jnp.float32)]
    scratch = (
        [pltpu.VMEM((_NCHK, _CH), jnp.int32)] * 2 +
        [pltpu.VMEM((_CH, _DH), jnp.float32)] * _NBUF +
        [pltpu.VMEM((_SRT, _DH), jnp.float32)] +
        [pltpu.VMEM_SHARED((_NACC, _DH), jnp.float32)] +
        [pltpu.SemaphoreType.DMA] * (2 * _NBUF)
    )
    if with_deg:
        out_types.append(jax.ShapeDtypeStruct((_NC * _NACC, _DEGW),
                                              jnp.float32))
        scratch += [
            pltpu.VMEM((_CH, _DEGW), jnp.float32),
            pltpu.VMEM((_SRT, _DEGW), jnp.float32),
            pltpu.VMEM_SHARED((_NACC, _DEGW), jnp.float32),
        ]

    @functools.partial(
        pl.kernel, mesh=mesh, out_type=out_types, scratch_types=scratch,
        compiler_params=pltpu.CompilerParams(use_tc_tiling_on_sc=False))
    def seg(vals, edge_r, *rest):
        if with_deg:
            (out, outd, src_v, dst_v, r0, r1, r2, r3, r4, zbuf, acc,
             g0, g1, g2, g3, g4, s0, s1, s2, s3, s4,
             ones_v, zbufd, accd) = rest
        else:
            (out, src_v, dst_v, r0, r1, r2, r3, r4, zbuf, acc,
             g0, g1, g2, g3, g4, s0, s1, s2, s3, s4) = rest
        rbufs = (r0, r1, r2, r3, r4)
        gsems = (g0, g1, g2, g3, g4)
        ssems = (s0, s1, s2, s3, s4)
        c = lax.axis_index("c")
        s = lax.axis_index("s")
        tid = c * _NS + s
        rowb = s * _RPT

        zero16 = jnp.zeros((16,), jnp.float32)

        pltpu.async_copy(edge_r.at[tid], src_v, gsems[0])
        pltpu.async_copy(edge_r.at[_NT + tid], dst_v, gsems[1])

        def zrow(i, _):
            for j in range(_DH // 16):
                zbuf[i, pl.ds(j * 16, 16)] = zero16
            return 0
        lax.fori_loop(0, _SRT, zrow, 0)
        for k in range(_RPT // _SRT):
            pltpu.async_copy(zbuf, acc.at[pl.ds(rowb + k * _SRT, _SRT)],
                             ssems[0])

        if with_deg:
            def zdrow(i, _):
                zbufd[i, pl.ds(0, 16)] = zero16
                return 0
            lax.fori_loop(0, _SRT, zdrow, 0)
            for k in range(_RPT // _SRT):
                pltpu.async_copy(zbufd, accd.at[pl.ds(rowb + k * _SRT, _SRT)],
                                 ssems[1])

            one16 = jnp.full((16,), 1.0, jnp.float32)

            def orow(i, _):
                ones_v[i, pl.ds(0, 16)] = one16
                return 0
            lax.fori_loop(0, _CH, orow, 0)

        for k in range(_RPT // _SRT):
            pltpu.make_async_copy(zbuf, acc.at[pl.ds(rowb, _SRT)],
                                  ssems[0]).wait()
            if with_deg:
                pltpu.make_async_copy(zbufd, accd.at[pl.ds(rowb, _SRT)],
                                      ssems[1]).wait()
        pltpu.make_async_copy(edge_r.at[tid], src_v, gsems[0]).wait()
        pltpu.make_async_copy(edge_r.at[_NT + tid], dst_v, gsems[1]).wait()

        plsc.subcore_barrier()

        for g in range(_GD):
            pltpu.async_copy(vals.at[src_v.at[g]], rbufs[g], gsems[g])

        def chunk(j, _):
            for b in range(_NBUF):
                @pl.when(j % _NBUF == b)
                def _(b=b):
                    nb = (b + _GD) % _NBUF
                    pltpu.make_async_copy(vals.at[src_v.at[j]], rbufs[b],
                                          gsems[b]).wait()
                    pltpu.async_copy(rbufs[b], acc.at[dst_v.at[j]],
                                     ssems[b], add=True)
                    if with_deg:
                        pltpu.async_copy(ones_v, accd.at[dst_v.at[j]],
                                         ssems[b], add=True)

                    @pl.when(j >= _SL)
                    def _():
                        pltpu.make_async_copy(rbufs[nb], acc.at[dst_v.at[j]],
                                              ssems[nb]).wait()
                        if with_deg:
                            pltpu.make_async_copy(
                                ones_v, accd.at[dst_v.at[j]],
                                ssems[nb]).wait()

                    @pl.when(j + _GD < _NCHK)
                    def _():
                        pltpu.async_copy(vals.at[src_v.at[j + _GD]],
                                         rbufs[nb], gsems[nb])
            return 0
        lax.fori_loop(0, _NCHK, chunk, 0)

        for jj in range(_NCHK - _SL, _NCHK):
            pltpu.make_async_copy(rbufs[jj % _NBUF], acc.at[dst_v.at[0]],
                                  ssems[jj % _NBUF]).wait()
            if with_deg:
                pltpu.make_async_copy(ones_v, accd.at[dst_v.at[0]],
                                      ssems[jj % _NBUF]).wait()

        plsc.subcore_barrier()

        ob = c * _NACC + rowb
        pltpu.async_copy(acc.at[pl.ds(rowb, _RPT)],
                         out.at[pl.ds(ob, _RPT)], gsems[0])
        if with_deg:
            pltpu.async_copy(accd.at[pl.ds(rowb, _RPT)],
                             outd.at[pl.ds(ob, _RPT)], gsems[1])
            pltpu.make_async_copy(accd.at[pl.ds(rowb, _RPT)],
                                  outd.at[pl.ds(ob, _RPT)], gsems[1]).wait()
        pltpu.make_async_copy(acc.at[pl.ds(rowb, _RPT)],
                              out.at[pl.ds(ob, _RPT)], gsems[0]).wait()

    return seg


def _tc_proj(x, W):
    def body(x_ref, w_ref, y_ref):
        y_ref[...] = jnp.dot(x_ref[...], w_ref[...],
                             preferred_element_type=jnp.float32)

    return pl.pallas_call(
        body,
        out_shape=jax.ShapeDtypeStruct((x.shape[0], W.shape[1]), jnp.float32),
    )(x, W)


def _tc_mid(p1, pd, z1, b1):
    def body(p_ref, pd_ref, z1_ref, b1_ref, h_ref, d_ref):
        agg = p_ref[0:_N, :] + p_ref[_NACC:_NACC + _N, :]
        deg = pd_ref[0:_N, 0:1] + pd_ref[_NACC:_NACC + _N, 0:1]
        degc = jnp.maximum(deg, 1.0)
        mean = agg / degc
        h = jnp.maximum(mean + b1_ref[...] + z1_ref[...], 0.0)
        h_ref[...] = h
        d_ref[...] = jnp.broadcast_to(degc, (_N, 8))

    return pl.pallas_call(
        body,
        out_shape=[jax.ShapeDtypeStruct((_N, _DH), jnp.float32),
                   jax.ShapeDtypeStruct((_N, 8), jnp.float32)],
    )(p1, pd, z1, b1)


def _tc_post(p2, degc, z2, b2, W2l):
    def body(p_ref, d_ref, z2_ref, b2_ref, w_ref, o_ref):
        agg = p_ref[0:_N, :] + p_ref[_NACC:_NACC + _N, :]
        mean = agg / d_ref[:, 0:1]
        o_ref[...] = (jnp.dot(mean, w_ref[...], preferred_element_type=jnp.float32)
                      + b2_ref[...] + z2_ref[...])

    return pl.pallas_call(
        body,
        out_shape=jax.ShapeDtypeStruct((_N, _DOUT), jnp.float32),
    )(p2, degc, z2, b2, W2l)


def _first(res):
    return res[0] if isinstance(res, (list, tuple)) else res


def kernel(x, edge_index, W1l, b1, W1r, W2l, b2, W2r):
    ar = jnp.arange(_EPAD - _E, dtype=jnp.int32) % 16
    padblk = jnp.stack([ar, _N + ar])
    edge_r = jnp.concatenate([edge_index, padblk], axis=1).reshape(
        2 * _NT, _NCHK, _CH)

    y1 = _tc_proj(x, W1l)
    p1, pd = _make_seg_sum(True)(y1, edge_r)
    z1 = _tc_proj(x, W1r)
    h, degc = _tc_mid(p1, pd, z1, b1.reshape(1, _DH))
    p2 = _first(_make_seg_sum(False)(h, edge_r))
    z2 = _tc_proj(h, W2r)
    out = _tc_post(p2, degc, z2, b2.reshape(1, _DOUT), W2l)
    return out


# speedup vs baseline: 1.0927x; 1.0034x over previous
import functools

import jax
import jax.numpy as jnp
from jax import lax
from jax.experimental import pallas as pl
from jax.experimental.pallas import tpu as pltpu
from jax.experimental.pallas import tpu_sc as plsc

_N = 10000
_E = 320000
_DIN = 128
_DH = 64
_DOUT = 128

_NC = 2
_NS = 16
_NT = _NC * _NS
_CH = 128
_NCHK = 79
_PER_TILE = _CH * _NCHK
_EPAD = _NT * _PER_TILE
_NACC = 10240
_RPT = _NACC // _NS
_SRT = 128
_DEGW = 16
_NBUF = 5
_GD = 4
_SL = _NBUF - _GD


@functools.lru_cache(maxsize=None)
def _make_seg_sum(with_deg):
    mesh = plsc.VectorSubcoreMesh(core_axis_name="c", subcore_axis_name="s")
    out_types = [jax.ShapeDtypeStruct((_NC * _NACC, _DH), jnp.float32)]
    scratch = (
        [pltpu.VMEM((_NCHK, _CH), jnp.int32)] * 2 +
        [pltpu.VMEM((_CH, _DH), jnp.float32)] * _NBUF +
        [pltpu.VMEM((_SRT, _DH), jnp.float32)] +
        [pltpu.VMEM_SHARED((_NACC, _DH), jnp.float32)] +
        [pltpu.SemaphoreType.DMA] * (2 * _NBUF)
    )
    if with_deg:
        out_types.append(jax.ShapeDtypeStruct((_NC * _NACC, _DEGW),
                                              jnp.float32))
        scratch += [
            pltpu.VMEM((_CH, _DEGW), jnp.float32),
            pltpu.VMEM((_SRT, _DEGW), jnp.float32),
            pltpu.VMEM_SHARED((_NACC, _DEGW), jnp.float32),
        ]

    @functools.partial(
        pl.kernel, mesh=mesh, out_type=out_types, scratch_types=scratch,
        compiler_params=pltpu.CompilerParams(use_tc_tiling_on_sc=False))
    def seg(vals, edge_r, *rest):
        if with_deg:
            (out, outd, src_v, dst_v, r0, r1, r2, r3, r4, zbuf, acc,
             g0, g1, g2, g3, g4, s0, s1, s2, s3, s4,
             ones_v, zbufd, accd) = rest
        else:
            (out, src_v, dst_v, r0, r1, r2, r3, r4, zbuf, acc,
             g0, g1, g2, g3, g4, s0, s1, s2, s3, s4) = rest
        rbufs = (r0, r1, r2, r3, r4)
        gsems = (g0, g1, g2, g3, g4)
        ssems = (s0, s1, s2, s3, s4)
        c = lax.axis_index("c")
        s = lax.axis_index("s")
        tid = c * _NS + s
        rowb = s * _RPT

        zero16 = jnp.zeros((16,), jnp.float32)

        pltpu.async_copy(edge_r.at[tid], src_v, gsems[0])
        pltpu.async_copy(edge_r.at[_NT + tid], dst_v, gsems[1])

        def zrow(i, _):
            for j in range(_DH // 16):
                zbuf[i, pl.ds(j * 16, 16)] = zero16
            return 0
        lax.fori_loop(0, _SRT, zrow, 0)
        for k in range(_RPT // _SRT):
            pltpu.async_copy(zbuf, acc.at[pl.ds(rowb + k * _SRT, _SRT)],
                             ssems[0])

        if with_deg:
            def zdrow(i, _):
                zbufd[i, pl.ds(0, 16)] = zero16
                return 0
            lax.fori_loop(0, _SRT, zdrow, 0)
            for k in range(_RPT // _SRT):
                pltpu.async_copy(zbufd, accd.at[pl.ds(rowb + k * _SRT, _SRT)],
                                 ssems[1])

            one16 = jnp.full((16,), 1.0, jnp.float32)

            def orow(i, _):
                ones_v[i, pl.ds(0, 16)] = one16
                return 0
            lax.fori_loop(0, _CH, orow, 0)

        for k in range(_RPT // _SRT):
            pltpu.make_async_copy(zbuf, acc.at[pl.ds(rowb, _SRT)],
                                  ssems[0]).wait()
            if with_deg:
                pltpu.make_async_copy(zbufd, accd.at[pl.ds(rowb, _SRT)],
                                      ssems[1]).wait()
        pltpu.make_async_copy(edge_r.at[tid], src_v, gsems[0]).wait()
        pltpu.make_async_copy(edge_r.at[_NT + tid], dst_v, gsems[1]).wait()

        plsc.subcore_barrier()

        for g in range(_GD):
            pltpu.async_copy(vals.at[src_v.at[g]], rbufs[g], gsems[g])

        def chunk(j, _):
            for b in range(_NBUF):
                @pl.when(j % _NBUF == b)
                def _(b=b):
                    nb = (b + _GD) % _NBUF
                    pltpu.make_async_copy(vals.at[src_v.at[j]], rbufs[b],
                                          gsems[b]).wait()
                    pltpu.async_copy(rbufs[b], acc.at[dst_v.at[j]],
                                     ssems[b], add=True)
                    if with_deg:
                        pltpu.async_copy(ones_v, accd.at[dst_v.at[j]],
                                         ssems[b], add=True)

                    @pl.when(j >= _SL)
                    def _():
                        pltpu.make_async_copy(rbufs[nb], acc.at[dst_v.at[j]],
                                              ssems[nb]).wait()
                        if with_deg:
                            pltpu.make_async_copy(
                                ones_v, accd.at[dst_v.at[j]],
                                ssems[nb]).wait()

                    @pl.when(j + _GD < _NCHK)
                    def _():
                        pltpu.async_copy(vals.at[src_v.at[j + _GD]],
                                         rbufs[nb], gsems[nb])
            return 0
        lax.fori_loop(0, _NCHK, chunk, 0)

        for jj in range(_NCHK - _SL, _NCHK):
            pltpu.make_async_copy(rbufs[jj % _NBUF], acc.at[dst_v.at[0]],
                                  ssems[jj % _NBUF]).wait()
            if with_deg:
                pltpu.make_async_copy(ones_v, accd.at[dst_v.at[0]],
                                      ssems[jj % _NBUF]).wait()

        plsc.subcore_barrier()

        ob = c * _NACC + rowb
        pltpu.async_copy(acc.at[pl.ds(rowb, _RPT)],
                         out.at[pl.ds(ob, _RPT)], gsems[0])
        if with_deg:
            pltpu.async_copy(accd.at[pl.ds(rowb, _RPT)],
                             outd.at[pl.ds(ob, _RPT)], gsems[1])
            pltpu.make_async_copy(accd.at[pl.ds(rowb, _RPT)],
                                  outd.at[pl.ds(ob, _RPT)], gsems[1]).wait()
        pltpu.make_async_copy(acc.at[pl.ds(rowb, _RPT)],
                              out.at[pl.ds(ob, _RPT)], gsems[0]).wait()

    return seg


def _tc_proj(x, W):
    def body(x_ref, w_ref, y_ref):
        y_ref[...] = jnp.dot(x_ref[...], w_ref[...],
                             preferred_element_type=jnp.float32)

    return pl.pallas_call(
        body,
        out_shape=jax.ShapeDtypeStruct((x.shape[0], W.shape[1]), jnp.float32),
    )(x, W)


def _tc_mid(p1, pd, z1, b1):
    def body(p_ref, pd_ref, z1_ref, b1_ref, h_ref, d_ref):
        agg = p_ref[0:_N, :] + p_ref[_NACC:_NACC + _N, :]
        deg = pd_ref[0:_N, 0:1] + pd_ref[_NACC:_NACC + _N, 0:1]
        degc = jnp.maximum(deg, 1.0)
        mean = agg / degc
        h = jnp.maximum(mean + b1_ref[...] + z1_ref[...], 0.0)
        h_ref[...] = h
        d_ref[...] = jnp.broadcast_to(degc, (_N, 8))

    return pl.pallas_call(
        body,
        out_shape=[jax.ShapeDtypeStruct((_N, _DH), jnp.float32),
                   jax.ShapeDtypeStruct((_N, 8), jnp.float32)],
    )(p1, pd, z1, b1)


def _tc_post(p2, degc, z2, b2, W2l):
    def body(p_ref, d_ref, z2_ref, b2_ref, w_ref, o_ref):
        agg = p_ref[0:_N, :] + p_ref[_NACC:_NACC + _N, :]
        mean = agg / d_ref[:, 0:1]
        o_ref[...] = (jnp.dot(mean, w_ref[...], preferred_element_type=jnp.float32)
                      + b2_ref[...] + z2_ref[...])

    return pl.pallas_call(
        body,
        out_shape=jax.ShapeDtypeStruct((_N, _DOUT), jnp.float32),
    )(p2, degc, z2, b2, W2l)


def _first(res):
    return res[0] if isinstance(res, (list, tuple)) else res


def kernel(x, edge_index, W1l, b1, W1r, W2l, b2, W2r):
    ar = jnp.arange(_EPAD - _E, dtype=jnp.int32) % 16
    padblk = jnp.stack([ar, _N + ar])
    edge_r = jnp.concatenate([edge_index, padblk], axis=1).reshape(
        2 * _NT, _NCHK, _CH)

    y1 = _tc_proj(x, W1l)
    p1, pd = _make_seg_sum(True)(y1, edge_r)
    z1 = _tc_proj(x, W1r)
    h, degc = _tc_mid(p1, pd, z1, b1.reshape(1, _DH))
    p2 = _first(_make_seg_sum(False)(h, edge_r))
    z2 = _tc_proj(h, W2r)
    out = _tc_post(p2, degc, z2, b2.reshape(1, _DOUT), W2l)
    return out
